# baseline (speedup 1.0000x reference)
import functools

import numpy as np
import jax
import jax.numpy as jnp
from jax import lax
from jax.experimental import pallas as pl
from jax.experimental.pallas import tpu as pltpu
from jax.experimental.pallas import tpu_sc as plsc

_B, _S, _V, _H, _G = 4, 2048, 50257, 768, 256
_NGOV = 7
_KGOV = _NGOV * _G
_GOV_SCALE = np.repeat(
    np.array([0.25, 0.25, 0.25, 0.25, 1.0, 1.0, 1.0], dtype=np.float32), _G
)

_N = _B * _S
_NC, _NS = 2, 16
_NW = _NC * _NS
_CH = 64

_NQ = 2
_NROWS_Q = _N // _NQ
_RPW = _NROWS_Q // _NW

_R = 512
_QSTEPS = _NROWS_Q // _R
_SBLK = _S // _R


def _sc_gather(ids_flat_q, token_table):
    mesh = plsc.VectorSubcoreMesh(core_axis_name="c", subcore_axis_name="s")

    @functools.partial(
        pl.kernel,
        out_type=jax.ShapeDtypeStruct((_NROWS_Q, _H), jnp.float32),
        mesh=mesh,
        scratch_types=[
            pltpu.VMEM((_CH,), jnp.int32),
            pltpu.VMEM((_CH, _H), jnp.float32),
            pltpu.SemaphoreType.DMA,
        ],
    )
    def gather_kernel(ids_hbm, table_hbm, out_hbm, idx_v, rows_v, sem):
        wid = lax.axis_index("s") * _NC + lax.axis_index("c")
        base = wid * _RPW
        for ci in range(_RPW // _CH):
            r0 = base + ci * _CH
            pltpu.sync_copy(ids_hbm.at[pl.ds(r0, _CH)], idx_v)
            pltpu.async_copy(table_hbm.at[idx_v], rows_v, sem).wait()
            pltpu.sync_copy(rows_v, out_hbm.at[pl.ds(r0, _CH)])

    return gather_kernel(ids_flat_q, token_table)


def _tc_body(*args):
    g_s = args[-1]
    out_ref = args[-2]
    y_ref, pos_ref, gov_ref, wr_ref, w_ref, b_ref, ga_ref, be_ref = args[-10:-2]

    @pl.when(pl.program_id(0) == 0)
    def _():
        c = gov_ref[:, :] * wr_ref[:, :]
        g_s[:, :] = (
            jnp.dot(c, w_ref[:, :], preferred_element_type=jnp.float32)
            + b_ref[:, :]
        )

    x = y_ref[:, :] + pos_ref[:, :] + g_s[:, :]
    mean = jnp.mean(x, axis=-1, keepdims=True)
    xc = x - mean
    var = jnp.mean(xc * xc, axis=-1, keepdims=True)
    o = xc / jnp.sqrt(var + 1e-5) * ga_ref[:, :] + be_ref[:, :]
    out_ref[:, :, :, :] = jnp.broadcast_to(o[None, None, :, :],
                                           (_B, 1, _R, _H))


def _tc_epilogue_q(q, out_prev, y_q, pos_table, govc, wrep, W, b2, g2, be2):
    data_specs = [
        pl.BlockSpec((_R, _H), lambda i: (i, 0)),
        pl.BlockSpec((_R, _H), lambda i: (i % _SBLK, 0)),
        pl.BlockSpec((1, _KGOV), lambda i: (0, 0)),
        pl.BlockSpec((1, _KGOV), lambda i: (0, 0)),
        pl.BlockSpec((_KGOV, _H), lambda i: (0, 0)),
        pl.BlockSpec((1, _H), lambda i: (0, 0)),
        pl.BlockSpec((1, _H), lambda i: (0, 0)),
        pl.BlockSpec((1, _H), lambda i: (0, 0)),
    ]
    b_per_q = _NROWS_Q // _S
    out_spec = pl.BlockSpec(
        (_B, 1, _R, _H),
        lambda i: (0, q * b_per_q + i // _SBLK, i % _SBLK, 0),
    )
    args = (y_q, pos_table, govc, wrep, W, b2, g2, be2)
    if q == 0:
        in_specs, aliases = data_specs, {}
    else:
        in_specs = [pl.BlockSpec(memory_space=pl.ANY)] + data_specs
        aliases = {0: 0}
        args = (out_prev,) + args
    return pl.pallas_call(
        _tc_body,
        grid=(_QSTEPS,),
        in_specs=in_specs,
        out_specs=out_spec,
        out_shape=jax.ShapeDtypeStruct((_B, _B, _S, _H), jnp.float32),
        scratch_shapes=[pltpu.VMEM((1, _H), jnp.float32)],
        input_output_aliases=aliases,
    )(*args)


def kernel(input_ids, token_table, pos_table, gov_tables, W, b, gamma, beta):
    ids_flat = input_ids.reshape(-1).astype(jnp.int32)
    govc = gov_tables.reshape(1, _KGOV)
    wrep = jnp.asarray(_GOV_SCALE).reshape(1, -1)
    b2 = b.reshape(1, -1)
    g2 = gamma.reshape(1, -1)
    be2 = beta.reshape(1, -1)

    ys = [
        _sc_gather(
            jax.lax.dynamic_slice(ids_flat, (q * _NROWS_Q,), (_NROWS_Q,)),
            token_table,
        )
        for q in range(_NQ)
    ]
    out = None
    for q in range(_NQ):
        out = _tc_epilogue_q(q, out, ys[q], pos_table, govc, wrep, W,
                             b2, g2, be2)
    return out


# speedup vs baseline: 1.0851x; 1.0851x over previous
import functools

import numpy as np
import jax
import jax.numpy as jnp
from jax import lax
from jax.experimental import pallas as pl
from jax.experimental.pallas import tpu as pltpu
from jax.experimental.pallas import tpu_sc as plsc

_B, _S, _V, _H, _G = 4, 2048, 50257, 768, 256
_NGOV = 7
_KGOV = _NGOV * _G
_GOV_SCALE = np.repeat(
    np.array([0.25, 0.25, 0.25, 0.25, 1.0, 1.0, 1.0], dtype=np.float32), _G
)

_N = _B * _S
_NC, _NS = 2, 16
_NW = _NC * _NS
_RPW = _N // _NW
_CH = 64
_NCH = _RPW // _CH

_R = 512
_SBLK = _S // _R


def _sc_gather(ids_flat, token_table):
    mesh = plsc.VectorSubcoreMesh(core_axis_name="c", subcore_axis_name="s")

    @functools.partial(
        pl.kernel,
        out_type=jax.ShapeDtypeStruct((_N, _H), jnp.float32),
        mesh=mesh,
        scratch_types=[
            pltpu.VMEM((_CH,), jnp.int32),
            pltpu.VMEM((_CH,), jnp.int32),
            pltpu.VMEM((_CH, _H), jnp.float32),
            pltpu.VMEM((_CH, _H), jnp.float32),
            pltpu.SemaphoreType.DMA,
            pltpu.SemaphoreType.DMA,
            pltpu.SemaphoreType.DMA,
            pltpu.SemaphoreType.DMA,
        ],
    )
    def gather_kernel(ids_hbm, table_hbm, out_hbm,
                      idx0, idx1, buf0, buf1, g0, g1, w0, w1):
        wid = lax.axis_index("s") * _NC + lax.axis_index("c")
        base = wid * _RPW
        idx = (idx0, idx1)
        buf = (buf0, buf1)
        gsem = (g0, g1)
        wsem = (w0, w1)

        pltpu.sync_copy(ids_hbm.at[pl.ds(base, _CH)], idx[0])
        gh = [None, None]
        wh = [None, None]
        gh[0] = pltpu.async_copy(table_hbm.at[idx[0]], buf[0], gsem[0])
        for ci in range(_NCH):
            cur = ci % 2
            nxt = 1 - cur
            gh[cur].wait()
            if ci + 1 < _NCH:
                if wh[nxt] is not None:
                    wh[nxt].wait()
                r1 = base + (ci + 1) * _CH
                pltpu.sync_copy(ids_hbm.at[pl.ds(r1, _CH)], idx[nxt])
                gh[nxt] = pltpu.async_copy(table_hbm.at[idx[nxt]], buf[nxt],
                                           gsem[nxt])
            r0 = base + ci * _CH
            wh[cur] = pltpu.async_copy(buf[cur], out_hbm.at[pl.ds(r0, _CH)],
                                       wsem[cur])
        wh[0].wait()
        wh[1].wait()

    return gather_kernel(ids_flat, token_table)


def _tc_epilogue(y, pos_table, govc, wrep, W, b2, gamma2, beta2):
    def body(y_ref, pos_ref, gov_ref, wr_ref, w_ref, b_ref, ga_ref, be_ref,
             out_ref, g_s):
        @pl.when((pl.program_id(0) == 0) & (pl.program_id(1) == 0))
        def _():
            c = gov_ref[:, :] * wr_ref[:, :]
            g_s[:, :] = (
                jnp.dot(c, w_ref[:, :], preferred_element_type=jnp.float32)
                + b_ref[:, :]
            )

        x = y_ref[:, :] + pos_ref[:, :] + g_s[:, :]
        mean = jnp.mean(x, axis=-1, keepdims=True)
        xc = x - mean
        var = jnp.mean(xc * xc, axis=-1, keepdims=True)
        o = xc / jnp.sqrt(var + 1e-5) * ga_ref[:, :] + be_ref[:, :]
        out_ref[:, :, :, :] = jnp.broadcast_to(o[None, None, :, :],
                                               (_B, 1, _R, _H))

    return pl.pallas_call(
        body,
        grid=(_SBLK, _B),
        in_specs=[
            pl.BlockSpec((_R, _H), lambda s, b: (b * _SBLK + s, 0)),
            pl.BlockSpec((_R, _H), lambda s, b: (s, 0)),
            pl.BlockSpec((1, _KGOV), lambda s, b: (0, 0)),
            pl.BlockSpec((1, _KGOV), lambda s, b: (0, 0)),
            pl.BlockSpec((_KGOV, _H), lambda s, b: (0, 0)),
            pl.BlockSpec((1, _H), lambda s, b: (0, 0)),
            pl.BlockSpec((1, _H), lambda s, b: (0, 0)),
            pl.BlockSpec((1, _H), lambda s, b: (0, 0)),
        ],
        out_specs=pl.BlockSpec((_B, 1, _R, _H),
                               lambda s, b: (0, b, s, 0)),
        out_shape=jax.ShapeDtypeStruct((_B, _B, _S, _H), jnp.float32),
        scratch_shapes=[pltpu.VMEM((1, _H), jnp.float32)],
    )(y, pos_table, govc, wrep, W, b2, gamma2, beta2)


def kernel(input_ids, token_table, pos_table, gov_tables, W, b, gamma, beta):
    ids_flat = input_ids.reshape(-1).astype(jnp.int32)
    y = _sc_gather(ids_flat, token_table)
    govc = gov_tables.reshape(1, _KGOV)
    wrep = jnp.asarray(_GOV_SCALE).reshape(1, -1)
    return _tc_epilogue(
        y, pos_table, govc, wrep, W,
        b.reshape(1, -1), gamma.reshape(1, -1), beta.reshape(1, -1),
    )
